# baseline (speedup 1.0000x reference)
import functools
import math

import jax
import jax.numpy as jnp
from jax import lax
from jax.experimental import pallas as pl
from jax.experimental.pallas import tpu as pltpu
from jax.experimental.pallas import tpu_sc as plsc

N = 10000
E = 320000
D = 128
G = 16
NF = 128
CUTOFF = 10.0

NC = 2
NS = 16
NW = NC * NS
EPW = E // NW
K = 40
NCH = EPW // K
NPT = 624
VPR = D // 16


def _cutoff_body(ew_ref, c_ref):
    c_ref[...] = 0.5 * (jnp.cos(ew_ref[...] * (math.pi / CUTOFF)) + 1.0)


def _tc_cutoff(edge_weight):
    ew = edge_weight.reshape(E // 128, 128)
    c = pl.pallas_call(
        _cutoff_body,
        out_shape=jax.ShapeDtypeStruct((E // 128, 128), jnp.float32),
    )(ew)
    return c.reshape(E, 1)


def _filter_body(attr_ref, c_ref, w1_ref, b1_ref, w2_ref, b2_ref, wf_ref):
    a1 = jnp.dot(attr_ref[...], w1_ref[...], preferred_element_type=jnp.float32)
    a1 = jnp.maximum(a1 + b1_ref[...], 0.0)
    wf = jnp.dot(a1, w2_ref[...], preferred_element_type=jnp.float32) + b2_ref[...]
    wf_ref[...] = wf * c_ref[...]


def _tc_filter(edge_attr, ew2, w1, b1, w2, b2):
    BE = 16000
    grid = (E // BE,)
    return pl.pallas_call(
        _filter_body,
        grid=grid,
        in_specs=[
            pl.BlockSpec((BE, G), lambda i: (i, 0)),
            pl.BlockSpec((BE, 1), lambda i: (i, 0)),
            pl.BlockSpec((G, NF), lambda i: (0, 0)),
            pl.BlockSpec((1, NF), lambda i: (0, 0)),
            pl.BlockSpec((NF, NF), lambda i: (0, 0)),
            pl.BlockSpec((1, NF), lambda i: (0, 0)),
        ],
        out_specs=pl.BlockSpec((BE, NF), lambda i: (i, 0)),
        out_shape=jax.ShapeDtypeStruct((E, NF), jnp.float32),
    )(edge_attr, ew2, w1, b1, w2, b2)


def _lin1_body(x_ref, w_ref, h_ref):
    h_ref[...] = jnp.dot(x_ref[...], w_ref[...], preferred_element_type=jnp.float32)


def _tc_lin1(x, lin1_w):
    return pl.pallas_call(
        _lin1_body,
        out_shape=jax.ShapeDtypeStruct((N, NF), jnp.float32),
    )(x, lin1_w)


def _sc_body(h_hbm, wf_hbm, src_hbm, dst_hbm, out_hbm,
             idx_v, rows_v, wfr_v, prod_v, agg_sh,
             isem0, isem1, isem2, isem3,
             gsem0, gsem1, wsem0, wsem1, ssem0, ssem1):
    cid = lax.axis_index("c")
    sid = lax.axis_index("s")
    wid = sid * NC + cid
    isem = (isem0, isem1, isem2, isem3)
    gsem = (gsem0, gsem1)
    wsem = (wsem0, wsem1)
    ssem = (ssem0, ssem1)

    def _zb(r, carry):
        for j in range(VPR):
            prod_v[0, r, pl.ds(j * 16, 16)] = jnp.zeros((16,), jnp.float32)
        return carry
    lax.fori_loop(0, K, _zb, 0)

    zbase = sid * NPT
    nz = NPT // K

    def _zc(i, carry):
        pltpu.sync_copy(prod_v.at[0], agg_sh.at[pl.ds(zbase + i * K, K)])
        return carry
    lax.fori_loop(0, nz, _zc, 0)
    zrem = NPT - nz * K

    @pl.when(sid == NS - 1)
    def _():
        for t in range((640 - NPT + zrem) // K):
            pltpu.sync_copy(prod_v.at[0],
                            agg_sh.at[pl.ds(zbase + nz * K + t * K, K)])

    if zrem:
        @pl.when(sid != NS - 1)
        def _():
            pltpu.sync_copy(prod_v.at[0, pl.ds(0, zrem)],
                            agg_sh.at[pl.ds(zbase + nz * K, zrem)])
    plsc.subcore_barrier()

    wbase = wid * EPW

    def _issue_idx(i, s4):
        base = wbase + i * K
        pltpu.async_copy(src_hbm.at[pl.ds(base, K)], idx_v.at[s4, 0], isem[s4])
        pltpu.async_copy(dst_hbm.at[pl.ds(base, K)], idx_v.at[s4, 1], isem[s4])

    def _issue_gw(i, s4, b):
        pltpu.make_async_copy(src_hbm.at[pl.ds(wbase, K)], idx_v.at[s4, 0],
                              isem[s4]).wait()
        pltpu.make_async_copy(dst_hbm.at[pl.ds(wbase, K)], idx_v.at[s4, 1],
                              isem[s4]).wait()
        pltpu.async_copy(h_hbm.at[idx_v.at[s4, 0]], rows_v.at[b], gsem[b])
        pltpu.async_copy(wf_hbm.at[pl.ds(wbase + i * K, K)], wfr_v.at[b],
                         wsem[b])

    def _wait_sc(s4, b):
        pltpu.make_async_copy(prod_v.at[b], agg_sh.at[idx_v.at[s4, 1]],
                              ssem[b]).wait()

    def _proc(i, s4, first=False, last=False):
        b = s4 % 2
        pltpu.make_async_copy(h_hbm.at[idx_v.at[s4, 0]], rows_v.at[b],
                              gsem[b]).wait()
        pltpu.make_async_copy(wf_hbm.at[pl.ds(wbase, K)], wfr_v.at[b],
                              wsem[b]).wait()
        if not first:
            _wait_sc((s4 + 2) % 4, b)
        if not last:
            _issue_idx(i + 2, (s4 + 2) % 4)

        def _mul(r, c2):
            for j in range(VPR):
                sl = pl.ds(j * 16, 16)
                prod_v[b, r, sl] = rows_v[b, r, sl] * wfr_v[b, r, sl]
            return c2
        lax.fori_loop(0, K, _mul, 0)

        pltpu.async_copy(prod_v.at[b], agg_sh.at[idx_v.at[s4, 1]], ssem[b],
                         add=True)

    _issue_idx(0, 0)
    _issue_idx(1, 1)
    _issue_gw(0, 0, 0)
    _issue_gw(1, 1, 1)
    _proc(0, 0, first=True)
    _issue_gw(2, 2, 0)
    _proc(1, 1, first=True)
    _issue_gw(3, 3, 1)

    def _quad(g, carry):
        i = 4 * g + 2
        for off in range(4):
            s4 = (2 + off) % 4
            _proc(i + off, s4)
            _issue_gw(i + off + 2, (s4 + 2) % 4, s4 % 2)
        return carry
    lax.fori_loop(0, (NCH - 6) // 4, _quad, 0)

    _proc(NCH - 4, 2)
    _issue_gw(NCH - 2, 0, 0)
    _proc(NCH - 3, 3)
    _issue_gw(NCH - 1, 1, 1)
    _proc(NCH - 2, 0, last=True)
    _proc(NCH - 1, 1, last=True)
    _wait_sc(0, 0)
    _wait_sc(1, 1)

    plsc.subcore_barrier()

    @pl.when(sid == NS - 1)
    def _():
        pltpu.sync_copy(agg_sh.at[pl.ds(zbase, 640)],
                        out_hbm.at[cid, pl.ds(zbase, 640)])

    @pl.when(sid != NS - 1)
    def _():
        pltpu.sync_copy(agg_sh.at[pl.ds(zbase, NPT)],
                        out_hbm.at[cid, pl.ds(zbase, NPT)])


def _sc_aggregate(h, wf, src, dst):
    mesh = plsc.VectorSubcoreMesh(core_axis_name="c", subcore_axis_name="s",
                                  num_cores=NC, num_subcores=NS)
    fn = pl.kernel(
        _sc_body,
        out_type=jax.ShapeDtypeStruct((NC, N, NF), jnp.float32),
        mesh=mesh,
        scratch_types=[
            pltpu.VMEM((4, 2, K), jnp.int32),
            pltpu.VMEM((2, K, NF), jnp.float32),
            pltpu.VMEM((2, K, NF), jnp.float32),
            pltpu.VMEM((2, K, NF), jnp.float32),
            pltpu.VMEM_SHARED((N, NF), jnp.float32),
            pltpu.SemaphoreType.DMA,
            pltpu.SemaphoreType.DMA,
            pltpu.SemaphoreType.DMA,
            pltpu.SemaphoreType.DMA,
            pltpu.SemaphoreType.DMA,
            pltpu.SemaphoreType.DMA,
            pltpu.SemaphoreType.DMA,
            pltpu.SemaphoreType.DMA,
            pltpu.SemaphoreType.DMA,
            pltpu.SemaphoreType.DMA,
        ],
    )
    return fn(h, wf, src, dst)


def _tail_body(a0_ref, a1_ref, x_ref, w2_ref, b2_ref, w_ref, b_ref, out_ref):
    a = a0_ref[...] + a1_ref[...]
    t = jnp.dot(a, w2_ref[...], preferred_element_type=jnp.float32) + b2_ref[...]
    t = jnp.maximum(t, 0.0)
    out_ref[...] = x_ref[...] + jnp.dot(t, w_ref[...],
                                        preferred_element_type=jnp.float32) + b_ref[...]


def _tc_tail(agg0, agg1, x, lin2_w, lin2_b, lin_w, lin_b):
    BN = 2000
    grid = (N // BN,)
    return pl.pallas_call(
        _tail_body,
        grid=grid,
        in_specs=[
            pl.BlockSpec((BN, NF), lambda i: (i, 0)),
            pl.BlockSpec((BN, NF), lambda i: (i, 0)),
            pl.BlockSpec((BN, D), lambda i: (i, 0)),
            pl.BlockSpec((NF, D), lambda i: (0, 0)),
            pl.BlockSpec((1, D), lambda i: (0, 0)),
            pl.BlockSpec((D, D), lambda i: (0, 0)),
            pl.BlockSpec((1, D), lambda i: (0, 0)),
        ],
        out_specs=pl.BlockSpec((BN, D), lambda i: (i, 0)),
        out_shape=jax.ShapeDtypeStruct((N, D), jnp.float32),
    )(agg0, agg1, x, lin2_w, lin2_b, lin_w, lin_b)


def kernel(x, conf_node_batch, edge_index_conf, edge_weight_conf, edge_attr_conf,
           edge_index_graph, edge_attr_graph,
           mlp_w1, mlp_b1, mlp_w2, mlp_b2, lin1_w, lin2_w, lin2_b, lin_w, lin_b):
    ew2 = _tc_cutoff(edge_weight_conf)
    wf = _tc_filter(edge_attr_conf, ew2, mlp_w1, mlp_b1.reshape(1, NF),
                    mlp_w2, mlp_b2.reshape(1, NF))
    h = _tc_lin1(x, lin1_w)
    partials = _sc_aggregate(h, wf, edge_index_conf[0], edge_index_conf[1])
    out = _tc_tail(partials[0], partials[1], x, lin2_w, lin2_b.reshape(1, D),
                   lin_w, lin_b.reshape(1, D))
    return out


# speedup vs baseline: 1.2628x; 1.2628x over previous
import functools
import math

import jax
import jax.numpy as jnp
from jax import lax
from jax.experimental import pallas as pl
from jax.experimental.pallas import tpu as pltpu
from jax.experimental.pallas import tpu_sc as plsc

N = 10000
E = 320000
D = 128
G = 16
NF = 128
CUTOFF = 10.0

NC = 2
NS = 16
NW = NC * NS
EPW = E // NW
K = 40
NCH = EPW // K
NPT = 624
VPR = D // 16


def _cutoff_body(ew_ref, c_ref):
    c_ref[...] = 0.5 * (jnp.cos(ew_ref[...] * (math.pi / CUTOFF)) + 1.0)


def _tc_cutoff(edge_weight):
    ew = edge_weight.reshape(E // 128, 128)
    c = pl.pallas_call(
        _cutoff_body,
        out_shape=jax.ShapeDtypeStruct((E // 128, 128), jnp.float32),
    )(ew)
    return c.reshape(E)


def _filter_body(attr_ref, c_ref, w1_ref, b1_ref, w2_ref, b2_ref, wf_ref):
    a1 = jnp.dot(attr_ref[...], w1_ref[...], preferred_element_type=jnp.float32)
    a1 = jnp.maximum(a1 + b1_ref[...], 0.0)
    wf = jnp.dot(a1, w2_ref[...], preferred_element_type=jnp.float32) + b2_ref[...]
    i = pl.program_id(0)
    cb = c_ref[pl.ds(i * attr_ref.shape[0], attr_ref.shape[0])]
    wf_ref[...] = wf * lax.broadcast_in_dim(cb, (attr_ref.shape[0], 1), (0,))


def _tc_filter(edge_attr, ew2, w1, b1, w2, b2):
    BE = 16000
    grid = (E // BE,)
    return pl.pallas_call(
        _filter_body,
        grid=grid,
        in_specs=[
            pl.BlockSpec((BE, G), lambda i: (i, 0)),
            pl.BlockSpec((E,), lambda i: (0,)),
            pl.BlockSpec((G, NF), lambda i: (0, 0)),
            pl.BlockSpec((1, NF), lambda i: (0, 0)),
            pl.BlockSpec((NF, NF), lambda i: (0, 0)),
            pl.BlockSpec((1, NF), lambda i: (0, 0)),
        ],
        out_specs=pl.BlockSpec((BE, NF), lambda i: (i, 0)),
        out_shape=jax.ShapeDtypeStruct((E, NF), jnp.float32),
    )(edge_attr, ew2, w1, b1, w2, b2)


def _lin1_body(x_ref, w_ref, h_ref):
    h_ref[...] = jnp.dot(x_ref[...], w_ref[...], preferred_element_type=jnp.float32)


def _tc_lin1(x, lin1_w):
    return pl.pallas_call(
        _lin1_body,
        out_shape=jax.ShapeDtypeStruct((N, NF), jnp.float32),
    )(x, lin1_w)


def _sc_body(h_hbm, wf_hbm, src_hbm, dst_hbm, out_hbm,
             idx_v, rows_v, wfr_v, prod_v, agg_sh,
             isem0, isem1, isem2, isem3,
             gsem0, gsem1, wsem0, wsem1, ssem0, ssem1):
    cid = lax.axis_index("c")
    sid = lax.axis_index("s")
    wid = sid * NC + cid
    isem = (isem0, isem1, isem2, isem3)
    gsem = (gsem0, gsem1)
    wsem = (wsem0, wsem1)
    ssem = (ssem0, ssem1)

    def _zb(r, carry):
        for j in range(VPR):
            prod_v[0, r, pl.ds(j * 16, 16)] = jnp.zeros((16,), jnp.float32)
        return carry
    lax.fori_loop(0, K, _zb, 0)

    zbase = sid * NPT
    nz = NPT // K

    def _zc(i, carry):
        pltpu.sync_copy(prod_v.at[0], agg_sh.at[pl.ds(zbase + i * K, K)])
        return carry
    lax.fori_loop(0, nz, _zc, 0)
    zrem = NPT - nz * K

    @pl.when(sid == NS - 1)
    def _():
        for t in range((640 - NPT + zrem) // K):
            pltpu.sync_copy(prod_v.at[0],
                            agg_sh.at[pl.ds(zbase + nz * K + t * K, K)])

    if zrem:
        @pl.when(sid != NS - 1)
        def _():
            pltpu.sync_copy(prod_v.at[0, pl.ds(0, zrem)],
                            agg_sh.at[pl.ds(zbase + nz * K, zrem)])
    plsc.subcore_barrier()

    wbase = wid * EPW

    def _issue_idx(i, s4):
        base = wbase + i * K
        pltpu.async_copy(src_hbm.at[pl.ds(base, K)], idx_v.at[s4, 0], isem[s4])
        pltpu.async_copy(dst_hbm.at[pl.ds(base, K)], idx_v.at[s4, 1], isem[s4])

    def _issue_gw(i, s4, b):
        pltpu.make_async_copy(src_hbm.at[pl.ds(wbase, K)], idx_v.at[s4, 0],
                              isem[s4]).wait()
        pltpu.make_async_copy(dst_hbm.at[pl.ds(wbase, K)], idx_v.at[s4, 1],
                              isem[s4]).wait()
        pltpu.async_copy(h_hbm.at[idx_v.at[s4, 0]], rows_v.at[b], gsem[b])
        pltpu.async_copy(wf_hbm.at[pl.ds(wbase + i * K, K)], wfr_v.at[b],
                         wsem[b])

    def _wait_sc(s4, b):
        pltpu.make_async_copy(prod_v.at[b], agg_sh.at[idx_v.at[s4, 1]],
                              ssem[b]).wait()

    def _proc(i, s4, first=False, last=False):
        b = s4 % 2
        pltpu.make_async_copy(h_hbm.at[idx_v.at[s4, 0]], rows_v.at[b],
                              gsem[b]).wait()
        pltpu.make_async_copy(wf_hbm.at[pl.ds(wbase, K)], wfr_v.at[b],
                              wsem[b]).wait()
        if not first:
            _wait_sc((s4 + 2) % 4, b)
        if not last:
            _issue_idx(i + 2, (s4 + 2) % 4)

        def _mul(r, c2):
            for j in range(VPR):
                sl = pl.ds(j * 16, 16)
                prod_v[b, r, sl] = rows_v[b, r, sl] * wfr_v[b, r, sl]
            return c2
        lax.fori_loop(0, K, _mul, 0)

        pltpu.async_copy(prod_v.at[b], agg_sh.at[idx_v.at[s4, 1]], ssem[b],
                         add=True)

    _issue_idx(0, 0)
    _issue_idx(1, 1)
    _issue_gw(0, 0, 0)
    _issue_gw(1, 1, 1)
    _proc(0, 0, first=True)
    _issue_gw(2, 2, 0)
    _proc(1, 1, first=True)
    _issue_gw(3, 3, 1)

    def _quad(g, carry):
        i = 4 * g + 2
        for off in range(4):
            s4 = (2 + off) % 4
            _proc(i + off, s4)
            _issue_gw(i + off + 2, (s4 + 2) % 4, s4 % 2)
        return carry
    lax.fori_loop(0, (NCH - 6) // 4, _quad, 0)

    _proc(NCH - 4, 2)
    _issue_gw(NCH - 2, 0, 0)
    _proc(NCH - 3, 3)
    _issue_gw(NCH - 1, 1, 1)
    _proc(NCH - 2, 0, last=True)
    _proc(NCH - 1, 1, last=True)
    _wait_sc(0, 0)
    _wait_sc(1, 1)

    plsc.subcore_barrier()

    @pl.when(sid == NS - 1)
    def _():
        pltpu.sync_copy(agg_sh.at[pl.ds(zbase, 640)],
                        out_hbm.at[cid, pl.ds(zbase, 640)])

    @pl.when(sid != NS - 1)
    def _():
        pltpu.sync_copy(agg_sh.at[pl.ds(zbase, NPT)],
                        out_hbm.at[cid, pl.ds(zbase, NPT)])


def _sc_aggregate(h, wf, src, dst):
    mesh = plsc.VectorSubcoreMesh(core_axis_name="c", subcore_axis_name="s",
                                  num_cores=NC, num_subcores=NS)
    fn = pl.kernel(
        _sc_body,
        out_type=jax.ShapeDtypeStruct((NC, N, NF), jnp.float32),
        mesh=mesh,
        scratch_types=[
            pltpu.VMEM((4, 2, K), jnp.int32),
            pltpu.VMEM((2, K, NF), jnp.float32),
            pltpu.VMEM((2, K, NF), jnp.float32),
            pltpu.VMEM((2, K, NF), jnp.float32),
            pltpu.VMEM_SHARED((N, NF), jnp.float32),
            pltpu.SemaphoreType.DMA,
            pltpu.SemaphoreType.DMA,
            pltpu.SemaphoreType.DMA,
            pltpu.SemaphoreType.DMA,
            pltpu.SemaphoreType.DMA,
            pltpu.SemaphoreType.DMA,
            pltpu.SemaphoreType.DMA,
            pltpu.SemaphoreType.DMA,
            pltpu.SemaphoreType.DMA,
            pltpu.SemaphoreType.DMA,
        ],
    )
    return fn(h, wf, src, dst)


def _tail_body(a0_ref, a1_ref, x_ref, w2_ref, b2_ref, w_ref, b_ref, out_ref):
    a = a0_ref[...] + a1_ref[...]
    t = jnp.dot(a, w2_ref[...], preferred_element_type=jnp.float32) + b2_ref[...]
    t = jnp.maximum(t, 0.0)
    out_ref[...] = x_ref[...] + jnp.dot(t, w_ref[...],
                                        preferred_element_type=jnp.float32) + b_ref[...]


def _tc_tail(agg0, agg1, x, lin2_w, lin2_b, lin_w, lin_b):
    BN = 2000
    grid = (N // BN,)
    return pl.pallas_call(
        _tail_body,
        grid=grid,
        in_specs=[
            pl.BlockSpec((BN, NF), lambda i: (i, 0)),
            pl.BlockSpec((BN, NF), lambda i: (i, 0)),
            pl.BlockSpec((BN, D), lambda i: (i, 0)),
            pl.BlockSpec((NF, D), lambda i: (0, 0)),
            pl.BlockSpec((1, D), lambda i: (0, 0)),
            pl.BlockSpec((D, D), lambda i: (0, 0)),
            pl.BlockSpec((1, D), lambda i: (0, 0)),
        ],
        out_specs=pl.BlockSpec((BN, D), lambda i: (i, 0)),
        out_shape=jax.ShapeDtypeStruct((N, D), jnp.float32),
    )(agg0, agg1, x, lin2_w, lin2_b, lin_w, lin_b)


def kernel(x, conf_node_batch, edge_index_conf, edge_weight_conf, edge_attr_conf,
           edge_index_graph, edge_attr_graph,
           mlp_w1, mlp_b1, mlp_w2, mlp_b2, lin1_w, lin2_w, lin2_b, lin_w, lin_b):
    ew2 = _tc_cutoff(edge_weight_conf)
    wf = _tc_filter(edge_attr_conf, ew2, mlp_w1, mlp_b1.reshape(1, NF),
                    mlp_w2, mlp_b2.reshape(1, NF))
    h = _tc_lin1(x, lin1_w)
    partials = _sc_aggregate(h, wf, edge_index_conf[0], edge_index_conf[1])
    out = _tc_tail(partials[0], partials[1], x, lin2_w, lin2_b.reshape(1, D),
                   lin_w, lin_b.reshape(1, D))
    return out
